# Initial kernel scaffold; baseline (speedup 1.0000x reference)
import jax
import jax.numpy as jnp
from jax.experimental import pallas as pl


def kernel(x, l1_w0, l1_w1, l1_gamma, l1_beta, l1_alpha, l20_w0, l20_w1, l20_gamma, l20_beta, l20_alpha, l2_w0, l2_wdw, l2_gamma, l2_beta, l2_alpha, bn2_gamma, bn2_beta, bn2_alpha, l30_w0, l30_w1, l30_gamma, l30_beta, l30_alpha, l3_w0, l3_wdw, l3_gamma, l3_beta, l3_alpha, bn3_gamma, bn3_beta, bn3_alpha, cls_w, cls_b):
    raise NotImplementedError("write your pallas kernel here")


# speedup vs baseline: 1.1738x; 1.1738x over previous
import functools

import jax
import jax.numpy as jnp
from jax import lax
from jax.experimental import pallas as pl
from jax.experimental.pallas import tpu as pltpu

_EPS = 1e-5
_DIL = (1, 2, 4, 8, 16, 32)
_WIN_BYTES = 12 * 1024 * 1024
_VMEM = 60 * 1024 * 1024


def _ru(x, m):
    return (x + m - 1) // m * m


def _cp(*sems):
    return pltpu.CompilerParams(dimension_semantics=tuple(sems),
                                vmem_limit_bytes=_VMEM)


def _coef(st, count, gamma, beta):
    mean = st[0] / count
    var = jnp.maximum(st[1] / count - mean * mean, 0.0)
    sc = gamma * lax.rsqrt(var + _EPS)
    return sc.reshape(1, -1), (beta - mean * sc).reshape(1, -1)


def _dilated_groups(p):
    g, k = {}, 0
    for d in _DIL:
        for i in range(3):
            for j in range(3):
                g.setdefault(p + (j - 1) * d, []).append((k, 0, p + (i - 1) * d))
                k += 1
    return tuple((dw, tuple(m)) for dw, m in sorted(g.items()))


def _dw5_groups():
    g = {}
    for i in range(5):
        for j in range(5):
            g.setdefault(j // 2, []).append((i * 5 + j, (i % 2) * 2 + (j % 2),
                                             i // 2))
    return tuple((dw, tuple(m)) for dw, m in sorted(g.items()))


def _rowpass_kernel(*refs, emit_y, emit_st, mm):
    i = 4 + (2 if mm else 0)
    x, sc, sh, al = refs[:4]
    outs = list(refs[i:])
    y = x[...] * sc[...] + sh[...]
    y = jnp.where(y >= 0.0, y, al[...] * y)
    if emit_y:
        outs.pop(0)[...] = y
    if mm:
        w, bb = refs[4], refs[5]
        outs.pop(0)[...] = (jnp.dot(y, w[...],
                                    preferred_element_type=jnp.float32)
                            + bb[...])
    if emit_st:
        st = outs.pop(0)
        st[0:1] = jnp.sum(y, axis=0, keepdims=True)
        st[1:2] = jnp.sum(y * y, axis=0, keepdims=True)


def _rowpass(x, sc, sh, al, w=None, b=None, emit_y=False, emit_st=False):
    m, c = x.shape
    tm = min(1024, m)
    nt = m // tm
    cvec = pl.BlockSpec((1, c), lambda i: (0, 0))
    in_specs = [pl.BlockSpec((tm, c), lambda i: (i, 0)), cvec, cvec, cvec]
    args = [x, sc, sh, al]
    out_shapes, out_specs = [], []
    if w is not None:
        cout = w.shape[1]
        if b is None:
            b = jnp.zeros((cout,), jnp.float32)
        in_specs += [pl.BlockSpec((c, cout), lambda i: (0, 0)),
                     pl.BlockSpec((1, cout), lambda i: (0, 0))]
        args += [w, b.reshape(1, cout)]
    if emit_y:
        out_shapes.append(jax.ShapeDtypeStruct((m, c), jnp.float32))
        out_specs.append(pl.BlockSpec((tm, c), lambda i: (i, 0)))
    if w is not None:
        out_shapes.append(jax.ShapeDtypeStruct((m, cout), jnp.float32))
        out_specs.append(pl.BlockSpec((tm, cout), lambda i: (i, 0)))
    if emit_st:
        out_shapes.append(jax.ShapeDtypeStruct((nt, 2, c), jnp.float32))
        out_specs.append(pl.BlockSpec((None, 2, c), lambda i: (i, 0, 0)))
    res = pl.pallas_call(
        functools.partial(_rowpass_kernel, emit_y=emit_y, emit_st=emit_st,
                          mm=w is not None),
        out_shape=tuple(out_shapes), grid=(nt,),
        in_specs=in_specs, out_specs=tuple(out_specs),
        compiler_params=_cp("parallel"))(*args)
    res = list(res)
    out = {}
    if emit_y:
        out["y"] = res.pop(0)
    if w is not None:
        out["mm"] = res.pop(0)
    if emit_st:
        out["st"] = jnp.sum(res.pop(0), axis=0)
    return out


_IDC = {}


def _identity_coef(c):
    if c not in _IDC:
        _IDC[c] = (jnp.ones((1, c), jnp.float32), jnp.zeros((1, c), jnp.float32),
                   jnp.ones((1, c), jnp.float32))
    return _IDC[c]


def _cat_mm_kernel(xa, xb, sa, ha, aa, sb, hb, ab, wa, wb, bb, o):
    A = xa[...] * sa[...] + ha[...]
    A = jnp.where(A >= 0.0, A, aa[...] * A)
    B = xb[...] * sb[...] + hb[...]
    B = jnp.where(B >= 0.0, B, ab[...] * B)
    o[...] = (jnp.dot(A, wa[...], preferred_element_type=jnp.float32)
              + jnp.dot(B, wb[...], preferred_element_type=jnp.float32)
              + bb[...])


def _cat_mm(xa, xb, ca, cb, wa, wb, bias):
    m, c = xa.shape
    cout = wa.shape[1]
    tm = min(1024, m)
    cvec = pl.BlockSpec((1, c), lambda i: (0, 0))
    y = pl.pallas_call(
        _cat_mm_kernel,
        out_shape=jax.ShapeDtypeStruct((m, cout), jnp.float32),
        grid=(m // tm,),
        in_specs=[pl.BlockSpec((tm, c), lambda i: (i, 0)),
                  pl.BlockSpec((tm, c), lambda i: (i, 0)),
                  cvec, cvec, cvec, cvec, cvec, cvec,
                  pl.BlockSpec((c, cout), lambda i: (0, 0)),
                  pl.BlockSpec((c, cout), lambda i: (0, 0)),
                  pl.BlockSpec((1, cout), lambda i: (0, 0))],
        out_specs=pl.BlockSpec((tm, cout), lambda i: (i, 0)),
        compiler_params=_cp("parallel"))(
            xa, xb, ca[0], ca[1], ca[2], cb[0], cb[1], cb[2], wa, wb,
            bias.reshape(1, cout))
    return y


def _mp_kernel(x_ref, o_ref, st_ref):
    m = jnp.max(x_ref[...], axis=0)
    o_ref[...] = m
    st_ref[0:1] = jnp.sum(m, axis=0, keepdims=True)
    st_ref[1:2] = jnp.sum(m * m, axis=0, keepdims=True)


def _maxpool(x):
    n, h, w, c = x.shape
    ho, wo = h // 2, w // 2
    parts = jnp.stack([x[:, a::2, b::2, :] for a in range(2) for b in range(2)])
    m = n * ho * wo
    x3 = parts.reshape(4, m, c)
    tm = min(1024, m)
    nt = m // tm
    y, st = pl.pallas_call(
        _mp_kernel,
        out_shape=(jax.ShapeDtypeStruct((m, c), jnp.float32),
                   jax.ShapeDtypeStruct((nt, 2, c), jnp.float32)),
        grid=(nt,),
        in_specs=[pl.BlockSpec((4, tm, c), lambda i: (0, i, 0))],
        out_specs=(pl.BlockSpec((tm, c), lambda i: (i, 0)),
                   pl.BlockSpec((None, 2, c), lambda i: (i, 0, 0))),
        compiler_params=_cp("parallel"))(x3)
    return y, jnp.sum(st, axis=0)


def _halo_dw_kernel(*refs, groups, halo, gh, h, res):
    if res:
        xs, r_ref, w_ref, o_ref, st_ref, buf, sem = refs
    else:
        xs, w_ref, o_ref, st_ref, buf, sem = refs
    th, wo, c = o_ref.shape
    b = pl.program_id(0)
    i = pl.program_id(1)
    sl = lax.rem(i, 2)

    def cp(t, s):
        return pltpu.make_async_copy(xs.at[b, :, pl.ds(t * th, th + halo)],
                                     buf.at[s], sem.at[s])

    @pl.when(i == 0)
    def _():
        cp(0, 0).start()

    @pl.when(i + 1 < gh)
    def _():
        cp(i + 1, 1 - sl).start()

    cp(i, sl).wait()

    acc = r_ref[...] if res else jnp.zeros((th, wo, c), jnp.float32)
    wv = w_ref[...]
    for dw, members in groups:
        gacc = None
        for (k, gp, dh) in members:
            t = wv[k].reshape(1, 1, c) * buf[sl, gp, dh:dh + th]
            gacc = t if gacc is None else gacc + t
        acc = acc + gacc[:, dw:dw + wo, :]
    o_ref[...] = acc

    rows = lax.broadcasted_iota(jnp.int32, (th, 1, 1), 0) + i * th
    av = jnp.where(rows < h, acc, 0.0)
    st_ref[0:1] = jnp.sum(av, axis=(0, 1)).reshape(1, c)
    st_ref[1:2] = jnp.sum(av * av, axis=(0, 1)).reshape(1, c)


def _halo_dw(xs, w, groups, halo, h, wo, residual=None):
    n, g, hin, wp, c = xs.shape
    per_row = g * wp * _ru(c, 128) * 4
    th = max(8, min(_ru(h, 8), (_WIN_BYTES // per_row - halo) // 8 * 8))
    gh = -(-h // th)
    need = gh * th + halo
    if need > hin:
        xs = jnp.pad(xs, ((0, 0), (0, 0), (0, need - hin), (0, 0), (0, 0)))
    in_specs = [pl.BlockSpec(memory_space=pl.ANY)]
    args = [xs]
    if residual is not None:
        in_specs.append(pl.BlockSpec((None, th, wo, c), lambda b, i: (b, i, 0, 0)))
        args.append(residual)
    in_specs.append(pl.BlockSpec(w.shape, lambda b, i: (0, 0)))
    args.append(w)
    out, st = pl.pallas_call(
        functools.partial(_halo_dw_kernel, groups=groups, halo=halo, gh=gh,
                          h=h, res=residual is not None),
        out_shape=(jax.ShapeDtypeStruct((n, h, wo, c), jnp.float32),
                   jax.ShapeDtypeStruct((n, gh, 2, c), jnp.float32)),
        grid=(n, gh),
        in_specs=in_specs,
        out_specs=(pl.BlockSpec((None, th, wo, c), lambda b, i: (b, i, 0, 0)),
                   pl.BlockSpec((None, None, 2, c), lambda b, i: (b, i, 0, 0))),
        scratch_shapes=[pltpu.VMEM((2, g, th + halo, wp, c), jnp.float32),
                        pltpu.SemaphoreType.DMA((2,))],
        compiler_params=_cp("parallel", "arbitrary"))(*args)
    return out, jnp.sum(st, axis=(0, 1))


def _parity4(x, pad):
    xp = jnp.pad(x, ((0, 0), (pad, pad), (pad, pad), (0, 0)))
    return jnp.stack([xp[:, a::2, b::2, :] for a in range(2) for b in range(2)],
                     axis=1)


def _dil_image_kernel(s_ref, sc, sh, al, w0, wd, o_ref, st_ref, pad_ref, *,
                      groups, p):
    hh, ww, c = o_ref.shape
    y = s_ref[...] * sc[...] + sh[...]
    y = jnp.where(y >= 0.0, y, al[...] * y)
    y0 = jnp.dot(y.reshape(hh * ww, c), w0[...],
                 preferred_element_type=jnp.float32)
    pad_ref[...] = jnp.zeros_like(pad_ref)
    pad_ref[p:p + hh, p:p + ww, :] = y0.reshape(hh, ww, c)
    wv = wd[...]
    acc = y
    for dw, members in groups:
        gacc = None
        for (k, _gp, dh) in members:
            t = wv[k].reshape(1, 1, c) * pad_ref[dh:dh + hh]
            gacc = t if gacc is None else gacc + t
        acc = acc + gacc[:, dw:dw + ww, :]
    o_ref[...] = acc
    st_ref[0:1] = jnp.sum(acc, axis=(0, 1)).reshape(1, c)
    st_ref[1:2] = jnp.sum(acc * acc, axis=(0, 1)).reshape(1, c)


def _dil_block_image(s, coef, w0, wd, p=32):
    n, hh, ww, c = s.shape
    sc, sh, al = (v.reshape(1, 1, c) for v in coef)
    cvec = pl.BlockSpec((1, 1, c), lambda b: (0, 0, 0))
    out, st = pl.pallas_call(
        functools.partial(_dil_image_kernel, groups=_dilated_groups(p), p=p),
        out_shape=(jax.ShapeDtypeStruct((n, hh, ww, c), jnp.float32),
                   jax.ShapeDtypeStruct((n, 2, c), jnp.float32)),
        grid=(n,),
        in_specs=[pl.BlockSpec((None, hh, ww, c), lambda b: (b, 0, 0, 0)),
                  cvec, cvec, cvec,
                  pl.BlockSpec((c, c), lambda b: (0, 0)),
                  pl.BlockSpec(wd.shape, lambda b: (0, 0))],
        out_specs=(pl.BlockSpec((None, hh, ww, c), lambda b: (b, 0, 0, 0)),
                   pl.BlockSpec((None, 2, c), lambda b: (b, 0, 0))),
        scratch_shapes=[pltpu.VMEM((hh + 2 * p, ww + 2 * p, c), jnp.float32)],
        compiler_params=_cp("parallel"))(s, sc, sh, al, w0, wd)
    return out, jnp.sum(st, axis=0)


@jax.jit
def _forward(x, l1_w0, l1_w1, l1_gamma, l1_beta, l1_alpha, l20_w0, l20_w1,
             l20_gamma, l20_beta, l20_alpha, l2_w0, l2_wdw, l2_gamma, l2_beta,
             l2_alpha, bn2_gamma, bn2_beta, bn2_alpha, l30_w0, l30_w1,
             l30_gamma, l30_beta, l30_alpha, l3_w0, l3_wdw, l3_gamma, l3_beta,
             l3_alpha, bn3_gamma, bn3_beta, bn3_alpha, cls_w, cls_b):
    xh = jnp.transpose(x, (0, 2, 3, 1)).astype(jnp.float32)
    n, H, W, _ = xh.shape
    g5 = _dw5_groups()

    idc3 = _identity_coef(3)
    y0 = _rowpass(xh.reshape(n * H * W, 3), *idc3, w=l1_w0)["mm"]
    y0 = y0.reshape(n, H, W, -1)
    yd1, std1 = _halo_dw(_parity4(y0, 2), l1_w1, g5, 2, H // 2, W // 2)
    mp1, stmp1 = _maxpool(xh)
    m1 = n * (H // 2) * (W // 2)
    yc1 = jnp.concatenate([yd1.reshape(m1, -1), mp1], axis=-1)
    st1 = jnp.concatenate([std1, stmp1], axis=-1)
    sc1, sh1 = _coef(st1, m1, l1_gamma, l1_beta)
    r = _rowpass(yc1, sc1, sh1, l1_alpha.reshape(1, -1), w=l20_w0, emit_y=True)
    o1, y0_20 = r["y"], r["mm"]

    h2, w2 = H // 4, W // 4
    yd20, std20 = _halo_dw(_parity4(y0_20.reshape(n, H // 2, W // 2, -1), 2),
                           l20_w1, g5, 2, h2, w2)
    mp20, stmp20 = _maxpool(o1.reshape(n, H // 2, W // 2, -1))
    m2 = n * h2 * w2
    yc20 = jnp.concatenate([yd20.reshape(m2, -1), mp20], axis=-1)
    st20in = jnp.concatenate([std20, stmp20], axis=-1)
    sc, sh = _coef(st20in, m2, l20_gamma, l20_beta)
    r = _rowpass(yc20, sc, sh, l20_alpha.reshape(1, -1), emit_y=True,
                 emit_st=True)
    o2_0, st20 = r["y"], r["st"]

    c2 = o2_0.shape[-1]
    gdil = _dilated_groups(32)
    s_prev, st_prev = None, None
    P = l2_w0.shape[0]
    for k in range(P):
        if k == 0:
            cf = _identity_coef(c2)
            r = _rowpass(o2_0, *cf, w=l2_w0[k])
            resid = o2_0
        else:
            sck, shk = _coef(st_prev, m2, l2_gamma[k - 1], l2_beta[k - 1])
            r = _rowpass(s_prev, sck, shk, l2_alpha[k - 1].reshape(1, -1),
                         w=l2_w0[k], emit_y=True)
            resid = r["y"]
        y0k = r["mm"].reshape(n, h2, w2, c2)
        xp = jnp.pad(y0k, ((0, 0), (32, 32), (32, 32), (0, 0)))[:, None]
        s_out, st_out = _halo_dw(xp, l2_wdw[k], gdil, 64, h2, w2,
                                 residual=resid.reshape(n, h2, w2, c2))
        s_prev, st_prev = s_out.reshape(m2, c2), st_out
    sck, shk = _coef(st_prev, m2, l2_gamma[P - 1], l2_beta[P - 1])
    r = _rowpass(s_prev, sck, shk, l2_alpha[P - 1].reshape(1, -1), emit_y=True,
                 emit_st=True)
    o2, st2 = r["y"], r["st"]

    stc2 = jnp.concatenate([st20, st2], axis=-1)
    sc2, sh2 = _coef(stc2, m2, bn2_gamma, bn2_beta)
    al2 = bn2_alpha.reshape(1, -1)
    y0_30 = _cat_mm(o2_0, o2,
                    (sc2[:, :c2], sh2[:, :c2], al2[:, :c2]),
                    (sc2[:, c2:], sh2[:, c2:], al2[:, c2:]),
                    l30_w0[:c2], l30_w0[c2:],
                    jnp.zeros((l30_w0.shape[1],), jnp.float32))

    h3, w3 = H // 8, W // 8
    m3 = n * h3 * w3
    c3 = l30_w0.shape[1]
    yd30, std30 = _halo_dw(_parity4(y0_30.reshape(n, h2, w2, c3), 2),
                           l30_w1, g5, 2, h3, w3)
    sc, sh = _coef(std30, m3, l30_gamma, l30_beta)
    r = _rowpass(yd30.reshape(m3, c3), sc, sh, l30_alpha.reshape(1, -1),
                 emit_y=True, emit_st=True)
    o3_0, st30 = r["y"], r["st"]

    Q = l3_w0.shape[0]
    s_img = o3_0.reshape(n, h3, w3, c3)
    st_prev = None
    for k in range(Q):
        if k == 0:
            cf = _identity_coef(c3)
        else:
            sck, shk = _coef(st_prev, m3, l3_gamma[k - 1], l3_beta[k - 1])
            cf = (sck, shk, l3_alpha[k - 1].reshape(1, -1))
        s_img, st_prev = _dil_block_image(s_img, cf, l3_w0[k], l3_wdw[k])
    sck, shk = _coef(st_prev, m3, l3_gamma[Q - 1], l3_beta[Q - 1])
    r = _rowpass(s_img.reshape(m3, c3), sck, shk,
                 l3_alpha[Q - 1].reshape(1, -1), emit_y=True, emit_st=True)
    o3, st3 = r["y"], r["st"]

    stc3 = jnp.concatenate([st30, st3], axis=-1)
    sc3, sh3 = _coef(stc3, m3, bn3_gamma, bn3_beta)
    al3 = bn3_alpha.reshape(1, -1)
    classes = cls_w.shape[1]
    cpad = _ru(classes, 128)
    wp = jnp.pad(cls_w, ((0, 0), (0, cpad - classes)))
    bp = jnp.pad(cls_b, ((0, cpad - classes),))
    cls = _cat_mm(o3_0, o3,
                  (sc3[:, :c3], sh3[:, :c3], al3[:, :c3]),
                  (sc3[:, c3:], sh3[:, c3:], al3[:, c3:]),
                  wp[:c3], wp[c3:], bp)[:, :classes]

    cls = cls.reshape(n, h3, w3, classes)
    up = jax.image.resize(cls, (n, H, W, classes), method="bilinear")
    return (jnp.transpose(up, (0, 3, 1, 2)),)


def kernel(*args):
    return _forward(*args)
